# baseline (speedup 1.0000x reference)
import functools

import jax
import jax.numpy as jnp
from jax.experimental import pallas as pl
from jax.experimental.pallas import tpu as pltpu

_P = 8
_NPACK = 12
_NREM = 4


def _body(x_ref, w8_ref, b8_ref, w4_ref, b4_ref, o_ref):
    ki = _P * 16
    ko = _P * 32
    xb = x_ref[...].astype(jnp.bfloat16)
    for p in range(_NPACK):
        o_ref[:, ko * p:ko * (p + 1)] = (
            jnp.dot(xb[:, ki * p:ki * (p + 1)], w8_ref[p],
                    preferred_element_type=jnp.float32)
            + b8_ref[p:p + 1, :]
        )
    o_ref[:, ko * _NPACK:] = (
        jnp.dot(xb[:, ki * _NPACK:], w4_ref[...],
                preferred_element_type=jnp.float32)
        + b4_ref[...]
    )


def _pack_blockdiag(Wg):
    G, P, I, O = Wg.shape
    eye = jnp.eye(P, dtype=Wg.dtype)
    return (Wg[:, :, :, None, :] * eye[None, :, None, :, None]).reshape(G, P * I, P * O)


@functools.partial(jax.jit, static_argnames=("block_b",))
def _attribute_embed(x, W, b, block_b=1024):
    B, N, I = x.shape
    O = W.shape[2]
    n_full = _NPACK * _P

    W8 = _pack_blockdiag(W[:n_full].reshape(_NPACK, _P, I, O)).astype(jnp.bfloat16)
    W4 = _pack_blockdiag(W[n_full:].reshape(1, _NREM, I, O))[0].astype(jnp.bfloat16)
    b8 = b[:n_full].reshape(_NPACK, _P * O)
    b4 = b[n_full:].reshape(1, _NREM * O)

    x2 = x.reshape(B, N * I)
    nb = B // block_b

    out2 = pl.pallas_call(
        _body,
        grid=(nb,),
        in_specs=[
            pl.BlockSpec((block_b, N * I), lambda i: (i, 0)),
            pl.BlockSpec(W8.shape, lambda i: (0, 0, 0)),
            pl.BlockSpec(b8.shape, lambda i: (0, 0)),
            pl.BlockSpec(W4.shape, lambda i: (0, 0)),
            pl.BlockSpec(b4.shape, lambda i: (0, 0)),
        ],
        out_specs=pl.BlockSpec((block_b, N * O), lambda i: (i, 0)),
        out_shape=jax.ShapeDtypeStruct((B, N * O), jnp.float32),
        compiler_params=pltpu.CompilerParams(
            dimension_semantics=("parallel",),
        ),
    )(x2, W8, b8, W4, b4)

    return out2.reshape(B, N, O)


def kernel(x, W, b):
    return _attribute_embed(x, W, b)


# speedup vs baseline: 2.5281x; 2.5281x over previous
import functools

import jax
import jax.numpy as jnp
from jax.experimental import pallas as pl
from jax.experimental.pallas import tpu as pltpu

_G = 8
_NPACK = 12
_NREM = 4


def _pack_blockdiag_t(Wg):
    P, G, I, O = Wg.shape
    Wt = Wg.transpose(0, 1, 3, 2)
    eye = jnp.eye(G, dtype=Wg.dtype)
    return (Wt[:, :, :, None, :] * eye[None, :, None, :, None]).reshape(P, G * O, G * I)


def _main_body(x_ref, w_ref, b_ref, o_ref):
    G, I, BB = x_ref.shape
    xb = x_ref[...].reshape(G * I, BB).astype(jnp.bfloat16)
    acc = jnp.dot(w_ref[0], xb, preferred_element_type=jnp.float32)
    o_ref[...] = (acc + b_ref[0]).reshape(G, 32, BB)


def _rem_body(x_ref, w_ref, b_ref, _y_ref, o_ref):
    G, I, BB = x_ref.shape
    xb = x_ref[...].reshape(G * I, BB).astype(jnp.bfloat16)
    acc = jnp.dot(w_ref[...], xb, preferred_element_type=jnp.float32)
    o_ref[...] = (acc + b_ref[...]).reshape(G, 32, BB)


@functools.partial(jax.jit, static_argnames=("block_b",))
def _attribute_embed(x, W, b, block_b=2048):
    B, N, I = x.shape
    O = W.shape[2]
    n_full = _NPACK * _G

    xt = x.transpose(1, 2, 0)

    W8 = _pack_blockdiag_t(W[:n_full].reshape(_NPACK, _G, I, O)).astype(jnp.bfloat16)
    W4 = _pack_blockdiag_t(W[n_full:].reshape(1, _NREM, I, O))[0].astype(jnp.bfloat16)
    b8 = b[:n_full].reshape(_NPACK, _G * O, 1)
    b4 = b[n_full:].reshape(_NREM * O, 1)

    nb = B // block_b

    y = pl.pallas_call(
        _main_body,
        grid=(_NPACK, nb),
        in_specs=[
            pl.BlockSpec((_G, I, block_b), lambda p, j: (p, 0, j)),
            pl.BlockSpec((1, _G * O, _G * I), lambda p, j: (p, 0, 0)),
            pl.BlockSpec((1, _G * O, 1), lambda p, j: (p, 0, 0)),
        ],
        out_specs=pl.BlockSpec((_G, O, block_b), lambda p, j: (p, 0, j)),
        out_shape=jax.ShapeDtypeStruct((N, O, B), jnp.float32),
        compiler_params=pltpu.CompilerParams(
            dimension_semantics=("parallel", "parallel"),
        ),
    )(xt, W8, b8)

    nrem_blocks = N // _NREM
    y = pl.pallas_call(
        _rem_body,
        grid=(nb,),
        in_specs=[
            pl.BlockSpec((_NREM, I, block_b), lambda j: (nrem_blocks - 1, 0, j)),
            pl.BlockSpec(W4.shape, lambda j: (0, 0)),
            pl.BlockSpec(b4.shape, lambda j: (0, 0)),
            pl.BlockSpec(memory_space=pltpu.MemorySpace.HBM),
        ],
        out_specs=pl.BlockSpec((_NREM, O, block_b), lambda j: (nrem_blocks - 1, 0, j)),
        out_shape=jax.ShapeDtypeStruct((N, O, B), jnp.float32),
        input_output_aliases={3: 0},
        compiler_params=pltpu.CompilerParams(
            dimension_semantics=("parallel",),
        ),
    )(xt, W4, b4, y)

    return y.transpose(2, 0, 1)


def kernel(x, W, b):
    return _attribute_embed(x, W, b)


# speedup vs baseline: 3.5901x; 1.4200x over previous
import functools

import jax
import jax.numpy as jnp
from jax.experimental import pallas as pl
from jax.experimental.pallas import tpu as pltpu

_G = 8
_NPACK = 13
_TB = 2048


def _pack_blockdiag_t(Wg):
    P, G, I, O = Wg.shape
    Wt = Wg.transpose(0, 1, 3, 2)
    eye = jnp.eye(G, dtype=Wg.dtype)
    return (Wt[:, :, :, None, :] * eye[None, :, None, :, None]).reshape(P, G * O, G * I)


def _body(x_ref, w_ref, b_ref, o_ref):
    G, I, B = x_ref.shape
    w = w_ref[0]
    bias = b_ref[0]
    for t in range(B // _TB):
        sl = slice(t * _TB, (t + 1) * _TB)
        xb = x_ref[:, :, sl].reshape(G * I, _TB).astype(jnp.bfloat16)
        acc = jnp.dot(w, xb, preferred_element_type=jnp.float32)
        o_ref[:, :, sl] = (acc + bias).reshape(G, 32, _TB)


@jax.jit
def _attribute_embed(x, W, b):
    B, N, I = x.shape
    O = W.shape[2]
    npad = _NPACK * _G - N

    xt = x.transpose(1, 2, 0)

    Wp = jnp.concatenate(
        [W, jnp.zeros((npad, I, O), W.dtype)], axis=0
    ).reshape(_NPACK, _G, I, O)
    W8 = _pack_blockdiag_t(Wp).astype(jnp.bfloat16)
    b8 = jnp.concatenate(
        [b, jnp.zeros((npad, O), b.dtype)], axis=0
    ).reshape(_NPACK, _G * O, 1)

    y = pl.pallas_call(
        _body,
        grid=(_NPACK,),
        in_specs=[
            pl.BlockSpec((_G, I, B), lambda p: (p, 0, 0)),
            pl.BlockSpec((1, _G * O, _G * I), lambda p: (p, 0, 0)),
            pl.BlockSpec((1, _G * O, 1), lambda p: (p, 0, 0)),
        ],
        out_specs=pl.BlockSpec((_G, O, B), lambda p: (p, 0, 0)),
        out_shape=jax.ShapeDtypeStruct((N, O, B), jnp.float32),
        compiler_params=pltpu.CompilerParams(
            dimension_semantics=("parallel",),
        ),
    )(xt, W8, b8)

    return y.transpose(2, 0, 1)


def kernel(x, W, b):
    return _attribute_embed(x, W, b)
